# Initial kernel scaffold; baseline (speedup 1.0000x reference)
import jax
import jax.numpy as jnp
from jax.experimental import pallas as pl


def kernel(points, xyz0, xyz1, xyz2, xyz3, xyz4, feats0, feats1, feats2, feats3, params):
    raise NotImplementedError("write your pallas kernel here")


# speedup vs baseline: 1.0004x; 1.0004x over previous
import jax
import jax.numpy as jnp
import numpy as np
from jax.experimental import pallas as pl

K = 16


def _linear(p, x):
    y = x @ p["w"]
    if "b" in p:
        y = y + p["b"]
    return y


def _bn_eval(p, x):
    return (x / jnp.sqrt(1.0 + 1e-5)) * p["gamma"] + p["beta"]


def _square_distance(src, dst):
    return jnp.sum((src[:, :, None, :] - dst[:, None, :, :]) ** 2, axis=-1)


def _index_points(points, idx):
    raw = idx.shape
    flat = idx.reshape(raw[0], -1)
    gi = jnp.broadcast_to(flat[:, :, None], (raw[0], flat.shape[1], points.shape[-1]))
    res = jnp.take_along_axis(points, gi, axis=1)
    return res.reshape(tuple(raw) + (points.shape[-1],))


def _transformer_block(p, xyz, features, k):
    dists = _square_distance(xyz, xyz)
    knn_idx = jnp.argsort(dists, axis=-1)[:, :, :k]
    knn_xyz = _index_points(xyz, knn_idx)
    pre = features
    x = _linear(p["fc1"], features)
    q = _linear(p["wq"], x)
    kk = _index_points(_linear(p["wk"], x), knn_idx)
    v = _index_points(_linear(p["wv"], x), knn_idx)
    d = xyz[:, :, None, :] - knn_xyz
    pos_enc = _linear(p["delta2"], jax.nn.relu(_linear(p["delta1"], d)))
    attn = _linear(p["gamma2"], jax.nn.relu(_linear(p["gamma1"], q[:, :, None, :] - kk + pos_enc)))
    attn = jax.nn.softmax(attn / np.sqrt(kk.shape[-1]), axis=-2)
    res = jnp.einsum("bmnf,bmnf->bmf", attn, v + pos_enc)
    res = _linear(p["fc2"], res) + pre
    return res


def _transition_up(p, xyz_coarse, points_coarse, xyz_fine, feats_fine):
    feats1 = jax.nn.relu(_bn_eval(p["bn1"], _linear(p["fc1"], points_coarse)))
    feats2 = jax.nn.relu(_bn_eval(p["bn2"], _linear(p["fc2"], feats_fine)))
    dists = _square_distance(xyz_fine, xyz_coarse)
    idx = jnp.argsort(dists, axis=-1)[:, :, :3]
    d3 = jnp.take_along_axis(dists, idx, axis=-1)
    dist_recip = 1.0 / (d3 + 1e-8)
    norm = jnp.sum(dist_recip, axis=2, keepdims=True)
    weight = dist_recip / norm
    interp = jnp.sum(_index_points(feats1, idx) * weight[..., None], axis=2)
    return interp + feats2


def _identity_kernel(x_ref, o_ref):
    o_ref[...] = x_ref[...]


def _pallas_identity(x):
    return pl.pallas_call(
        _identity_kernel,
        out_shape=jax.ShapeDtypeStruct(x.shape, x.dtype),
    )(x)


def kernel(points, xyz0, xyz1, xyz2, xyz3, xyz4, feats0, feats1, feats2, feats3, params):
    xyzs = [xyz0, xyz1, xyz2, xyz3, xyz4]
    featss = [feats0, feats1, feats2, feats3]
    xyz = xyzs[-1]
    f = params["fc"]
    h = _linear(f["l3"], jax.nn.relu(_linear(f["l2"], jax.nn.relu(_linear(f["l1"], points)))))
    pts = _transformer_block(params["transformer"], xyz, h, K)
    for i in range(4):
        fine_xyz = xyzs[3 - i]
        fine_feats = featss[3 - i]
        pts = _transition_up(params["tu"][i], xyz, pts, fine_xyz, fine_feats)
        xyz = fine_xyz
        pts = _transformer_block(params["tr"][i], xyz, pts, K)
    return _pallas_identity(pts)


# speedup vs baseline: 1656.3734x; 1655.6842x over previous
import functools

import jax
import jax.numpy as jnp
import numpy as np
from jax.experimental import pallas as pl
from jax.experimental.pallas import tpu as pltpu
from jax.experimental.pallas import tpu_sc as plsc

K = 16
DM = 256
XP = 32
TD = 3 * DM
NC, NS = 2, 16
NW = NC * NS
BIGF = np.float32(1e30)
BIGI = np.int32(1 << 30)
BN_INV = np.float32(1.0 / np.sqrt(1.0 + 1e-5))
SM_SCALE = np.float32(1.0 / 16.0)


def _knn_body(q_ref, xt_ref, o_ref, *, k):
    qb = q_ref[...]
    xt = xt_ref[...]
    d = (jnp.sum(qb * qb, axis=1, keepdims=True)
         + jnp.sum(xt * xt, axis=0, keepdims=True)
         - 2.0 * jnp.dot(qb, xt, preferred_element_type=jnp.float32,
                         precision=jax.lax.Precision.HIGHEST))
    cols = jax.lax.broadcasted_iota(jnp.int32, d.shape, 1)
    outs = []
    for _ in range(k):
        m = jnp.min(d, axis=1, keepdims=True)
        idx = jnp.min(jnp.where(d <= m, cols, BIGI), axis=1, keepdims=True)
        outs.append(idx)
        d = jnp.where(cols == idx, BIGF, d)
    o_ref[...] = jnp.concatenate(outs, axis=1)


def _knn(xyz8, xyz8t, k):
    n = xyz8.shape[0]
    m = min(256, n)
    return pl.pallas_call(
        functools.partial(_knn_body, k=k),
        grid=(n // m,),
        in_specs=[
            pl.BlockSpec((m, 8), lambda i: (i, 0)),
            pl.BlockSpec((8, n), lambda i: (0, 0)),
        ],
        out_specs=pl.BlockSpec((m, k), lambda i: (i, 0)),
        out_shape=jax.ShapeDtypeStruct((n, k), jnp.int32),
        compiler_params=pltpu.CompilerParams(dimension_semantics=("parallel",)),
    )(xyz8, xyz8t)


def _pre_body(f_ref, xyz_ref, wf_ref, bf_ref, wq_ref, wk_ref, wv_ref, wd1_ref,
              q_ref, e_ref, t_ref):
    x = jnp.dot(f_ref[...], wf_ref[...],
                preferred_element_type=jnp.float32) + bf_ref[...]
    q_ref[...] = jnp.dot(x, wq_ref[...], preferred_element_type=jnp.float32)
    e = jnp.dot(xyz_ref[...], wd1_ref[...], preferred_element_type=jnp.float32)
    e_ref[...] = e
    t_ref[:, 0:DM] = jnp.dot(x, wk_ref[...], preferred_element_type=jnp.float32)
    t_ref[:, DM:2 * DM] = jnp.dot(x, wv_ref[...],
                                  preferred_element_type=jnp.float32)
    t_ref[:, 2 * DM:TD] = e


def _pre(feats, xyz32, wf, bf, wq, wk, wv, wd1):
    n, c = feats.shape
    m = min(512, n)
    return pl.pallas_call(
        _pre_body,
        grid=(n // m,),
        in_specs=[
            pl.BlockSpec((m, c), lambda i: (i, 0)),
            pl.BlockSpec((m, XP), lambda i: (i, 0)),
            pl.BlockSpec((c, DM), lambda i: (0, 0)),
            pl.BlockSpec((1, DM), lambda i: (0, 0)),
            pl.BlockSpec((DM, DM), lambda i: (0, 0)),
            pl.BlockSpec((DM, DM), lambda i: (0, 0)),
            pl.BlockSpec((DM, DM), lambda i: (0, 0)),
            pl.BlockSpec((XP, DM), lambda i: (0, 0)),
        ],
        out_specs=[
            pl.BlockSpec((m, DM), lambda i: (i, 0)),
            pl.BlockSpec((m, DM), lambda i: (i, 0)),
            pl.BlockSpec((m, TD), lambda i: (i, 0)),
        ],
        out_shape=[
            jax.ShapeDtypeStruct((n, DM), jnp.float32),
            jax.ShapeDtypeStruct((n, DM), jnp.float32),
            jax.ShapeDtypeStruct((n, TD), jnp.float32),
        ],
        compiler_params=pltpu.CompilerParams(dimension_semantics=("parallel",)),
    )(feats, xyz32, wf, bf, wq, wk, wv, wd1)


def _make_sc_gather(n, b):
    bpw = b // NW
    ch = min(bpw, 64)
    nloop = bpw // ch
    mesh = plsc.VectorSubcoreMesh(core_axis_name="c", subcore_axis_name="s",
                                  num_cores=NC, num_subcores=NS)

    def body(tab_ref, idx_ref, out_ref, idx_v, rows_v, sem):
        wid = jax.lax.axis_index("s") * NC + jax.lax.axis_index("c")
        base = wid * bpw

        def step(i, carry):
            off = base + i * ch
            pltpu.sync_copy(idx_ref.at[pl.ds(off, ch)], idx_v)
            pltpu.async_copy(tab_ref.at[idx_v], rows_v, sem).wait()
            pltpu.sync_copy(rows_v, out_ref.at[pl.ds(off, ch)])
            return carry

        jax.lax.fori_loop(0, nloop, step, 0)

    return pl.kernel(
        body,
        out_type=jax.ShapeDtypeStruct((b, TD), jnp.float32),
        mesh=mesh,
        scratch_types=[
            pltpu.VMEM((ch,), jnp.int32),
            pltpu.VMEM((ch, TD), jnp.float32),
            pltpu.SemaphoreType.DMA,
        ],
    )


def _sc_gather(table, flat_idx):
    n = table.shape[0]
    b = flat_idx.shape[0]
    return _make_sc_gather(n, b)(table, flat_idx)


def _attn_body(g_ref, q_ref, eq_ref, pre_ref, bd1, wd2, bd2,
               wg1, bg1, wg2, bg2, wo, bo, o_ref, *, m):
    kk = g_ref[:, 0:DM]
    vv = g_ref[:, DM:2 * DM]
    ee = g_ref[:, 2 * DM:TD]
    q3 = jnp.broadcast_to(q_ref[...][:, None, :], (m, K, DM))
    e3 = jnp.broadcast_to(eq_ref[...][:, None, :], (m, K, DM))
    p1 = jnp.maximum(e3.reshape(m * K, DM) - ee + bd1[...], 0.0)
    pos = jnp.dot(p1, wd2[...], preferred_element_type=jnp.float32) + bd2[...]
    t = q3.reshape(m * K, DM) - kk + pos
    a1 = jnp.maximum(
        jnp.dot(t, wg1[...], preferred_element_type=jnp.float32) + bg1[...],
        0.0)
    lg = (jnp.dot(a1, wg2[...], preferred_element_type=jnp.float32)
          + bg2[...]) * SM_SCALE
    lg3 = lg.reshape(m, K, DM)
    mx = jnp.max(lg3, axis=1, keepdims=True)
    e = jnp.exp(lg3 - mx)
    s = jnp.sum(e, axis=1, keepdims=True)
    vp = (vv + pos).reshape(m, K, DM)
    r = jnp.sum((e / s) * vp, axis=1)
    o_ref[...] = (jnp.dot(r, wo[...], preferred_element_type=jnp.float32)
                  + bo[...] + pre_ref[...])


def _attn(g, q, eq, pre, bd1, wd2, bd2, wg1, bg1, wg2, bg2, wo, bo):
    n, c = pre.shape
    m = min(128, n)
    return pl.pallas_call(
        functools.partial(_attn_body, m=m),
        grid=(n // m,),
        in_specs=[
            pl.BlockSpec((m * K, TD), lambda i: (i, 0)),
            pl.BlockSpec((m, DM), lambda i: (i, 0)),
            pl.BlockSpec((m, DM), lambda i: (i, 0)),
            pl.BlockSpec((m, c), lambda i: (i, 0)),
            pl.BlockSpec((1, DM), lambda i: (0, 0)),
            pl.BlockSpec((DM, DM), lambda i: (0, 0)),
            pl.BlockSpec((1, DM), lambda i: (0, 0)),
            pl.BlockSpec((DM, DM), lambda i: (0, 0)),
            pl.BlockSpec((1, DM), lambda i: (0, 0)),
            pl.BlockSpec((DM, DM), lambda i: (0, 0)),
            pl.BlockSpec((1, DM), lambda i: (0, 0)),
            pl.BlockSpec((DM, c), lambda i: (0, 0)),
            pl.BlockSpec((1, c), lambda i: (0, 0)),
        ],
        out_specs=pl.BlockSpec((m, c), lambda i: (i, 0)),
        out_shape=jax.ShapeDtypeStruct((n, c), jnp.float32),
        compiler_params=pltpu.CompilerParams(dimension_semantics=("parallel",)),
    )(g, q, eq, pre, bd1, wd2, bd2, wg1, bg1, wg2, bg2, wo, bo)


def _tu_body(xf_ref, xct_ref, pc_ref, ff_ref, w1, b1, g1, t1,
             w2, b2, g2, t2, o_ref):
    f2 = jnp.maximum(
        (jnp.dot(ff_ref[...], w2[...], preferred_element_type=jnp.float32)
         + b2[...]) * (g2[...] * BN_INV) + t2[...], 0.0)
    f1 = jnp.maximum(
        (jnp.dot(pc_ref[...], w1[...], preferred_element_type=jnp.float32)
         + b1[...]) * (g1[...] * BN_INV) + t1[...], 0.0)
    qb = xf_ref[...]
    xt = xct_ref[...]
    d = (jnp.sum(qb * qb, axis=1, keepdims=True)
         + jnp.sum(xt * xt, axis=0, keepdims=True)
         - 2.0 * jnp.dot(qb, xt, preferred_element_type=jnp.float32,
                         precision=jax.lax.Precision.HIGHEST))
    d = jnp.maximum(d, 0.0)
    cols = jax.lax.broadcasted_iota(jnp.int32, d.shape, 1)
    recs, sels = [], []
    for _ in range(3):
        mn = jnp.min(d, axis=1, keepdims=True)
        idx = jnp.min(jnp.where(d <= mn, cols, BIGI), axis=1, keepdims=True)
        recs.append(1.0 / (mn + 1e-8))
        sels.append(cols == idx)
        d = jnp.where(cols == idx, BIGF, d)
    norm = recs[0] + recs[1] + recs[2]
    w = (jnp.where(sels[0], recs[0] / norm, 0.0)
         + jnp.where(sels[1], recs[1] / norm, 0.0)
         + jnp.where(sels[2], recs[2] / norm, 0.0))
    interp = jnp.dot(w, f1, preferred_element_type=jnp.float32)
    o_ref[...] = interp + f2


def _tu(p, xyzf8, xyzc8t, points_coarse, feats_fine):
    nf, c = feats_fine.shape
    s = points_coarse.shape[0]
    mf = min(256, nf)
    c2 = points_coarse.shape[1]
    r2 = lambda a: a.reshape(1, -1)
    return pl.pallas_call(
        _tu_body,
        grid=(nf // mf,),
        in_specs=[
            pl.BlockSpec((mf, 8), lambda i: (i, 0)),
            pl.BlockSpec((8, s), lambda i: (0, 0)),
            pl.BlockSpec((s, c2), lambda i: (0, 0)),
            pl.BlockSpec((mf, c), lambda i: (i, 0)),
        ] + [pl.BlockSpec(bs, lambda i: (0, 0)) for bs in
             [(c2, c), (1, c), (1, c), (1, c),
              (c, c), (1, c), (1, c), (1, c)]],
        out_specs=pl.BlockSpec((mf, c), lambda i: (i, 0)),
        out_shape=jax.ShapeDtypeStruct((nf, c), jnp.float32),
        compiler_params=pltpu.CompilerParams(dimension_semantics=("parallel",)),
    )(xyzf8, xyzc8t, points_coarse, feats_fine,
      p["fc1"]["w"], r2(p["fc1"]["b"]), r2(p["bn1"]["gamma"]), r2(p["bn1"]["beta"]),
      p["fc2"]["w"], r2(p["fc2"]["b"]), r2(p["bn2"]["gamma"]), r2(p["bn2"]["beta"]))


def _mlp_body(x_ref, w1, b1, w2, b2, w3, b3, o_ref):
    h = jnp.maximum(jnp.dot(x_ref[...], w1[...],
                            preferred_element_type=jnp.float32) + b1[...], 0.0)
    h = jnp.maximum(jnp.dot(h, w2[...],
                            preferred_element_type=jnp.float32) + b2[...], 0.0)
    o_ref[...] = jnp.dot(h, w3[...],
                         preferred_element_type=jnp.float32) + b3[...]


def _mlp(x, f):
    n, c = x.shape
    r2 = lambda a: a.reshape(1, -1)
    return pl.pallas_call(
        _mlp_body,
        out_shape=jax.ShapeDtypeStruct((n, c), jnp.float32),
    )(x, f["l1"]["w"], r2(f["l1"]["b"]), f["l2"]["w"], r2(f["l2"]["b"]),
      f["l3"]["w"], r2(f["l3"]["b"]))


def _tb(p, xyz8, xyz8t, xyz32, feats):
    idx = _knn(xyz8, xyz8t, K)
    r2 = lambda a: a.reshape(1, -1)
    wd1 = jnp.pad(p["delta1"]["w"], ((0, XP - 3), (0, 0)))
    q, e, table = _pre(feats, xyz32, p["fc1"]["w"], r2(p["fc1"]["b"]),
                       p["wq"]["w"], p["wk"]["w"], p["wv"]["w"], wd1)
    g = _sc_gather(table, idx.reshape(-1))
    return _attn(g, q, e, feats,
                 r2(p["delta1"]["b"]), p["delta2"]["w"], r2(p["delta2"]["b"]),
                 p["gamma1"]["w"], r2(p["gamma1"]["b"]),
                 p["gamma2"]["w"], r2(p["gamma2"]["b"]),
                 p["fc2"]["w"], r2(p["fc2"]["b"]))


def kernel(points, xyz0, xyz1, xyz2, xyz3, xyz4, feats0, feats1, feats2,
           feats3, params):
    xyzs = [x[0] for x in (xyz0, xyz1, xyz2, xyz3, xyz4)]
    featss = [f[0] for f in (feats0, feats1, feats2, feats3)]
    pad8 = [jnp.pad(x, ((0, 0), (0, 8 - 3))) for x in xyzs]
    pad8t = [x.T for x in pad8]
    pad32 = [jnp.pad(x, ((0, 0), (0, XP - 3))) for x in xyzs]

    h = _mlp(points[0], params["fc"])
    lvl = 4
    pts = _tb(params["transformer"], pad8[lvl], pad8t[lvl], pad32[lvl], h)
    for i in range(4):
        fine = 3 - i
        pts = _tu(params["tu"][i], pad8[fine], pad8t[lvl],
                  pts, featss[fine])
        lvl = fine
        pts = _tb(params["tr"][i], pad8[lvl], pad8t[lvl], pad32[lvl], pts)
    return pts[None]


# speedup vs baseline: 1833.5206x; 1.1069x over previous
import functools

import jax
import jax.numpy as jnp
import numpy as np
from jax.experimental import pallas as pl
from jax.experimental.pallas import tpu as pltpu
from jax.experimental.pallas import tpu_sc as plsc

K = 16
DM = 256
XP = 32
TD = 3 * DM
NC, NS = 2, 16
NW = NC * NS
BIGF = np.float32(1e30)
BIGI = np.int32(1 << 30)
BN_INV = np.float32(1.0 / np.sqrt(1.0 + 1e-5))
SM_SCALE = np.float32(1.0 / 16.0)


def _knn_body(q_ref, xt_ref, o_ref, *, k):
    qb = q_ref[...]
    xt = xt_ref[...]
    d = (jnp.sum(qb * qb, axis=1, keepdims=True)
         + jnp.sum(xt * xt, axis=0, keepdims=True)
         - 2.0 * jnp.dot(qb, xt, preferred_element_type=jnp.float32,
                         precision=jax.lax.Precision.HIGHEST))
    cols = jax.lax.broadcasted_iota(jnp.int32, d.shape, 1).astype(jnp.float32)
    outs = []
    for _ in range(k):
        m = jnp.min(d, axis=1, keepdims=True)
        idx = jnp.min(jnp.where(d <= m, cols, BIGF), axis=1, keepdims=True)
        outs.append(idx)
        d = jnp.where(cols == idx, BIGF, d)
    o_ref[...] = jnp.concatenate(outs, axis=1).astype(jnp.int32)


def _knn(xyz8, xyz8t, k):
    n = xyz8.shape[0]
    m = min(256, n)
    return pl.pallas_call(
        functools.partial(_knn_body, k=k),
        grid=(n // m,),
        in_specs=[
            pl.BlockSpec((m, 8), lambda i: (i, 0)),
            pl.BlockSpec((8, n), lambda i: (0, 0)),
        ],
        out_specs=pl.BlockSpec((m, k), lambda i: (i, 0)),
        out_shape=jax.ShapeDtypeStruct((n, k), jnp.int32),
        compiler_params=pltpu.CompilerParams(dimension_semantics=("parallel",)),
    )(xyz8, xyz8t)


def _pre_body(f_ref, xyz_ref, wf_ref, bf_ref, wq_ref, wk_ref, wv_ref, wd1_ref,
              q_ref, e_ref, t_ref):
    x = jnp.dot(f_ref[...], wf_ref[...],
                preferred_element_type=jnp.float32) + bf_ref[...]
    q_ref[...] = jnp.dot(x, wq_ref[...], preferred_element_type=jnp.float32)
    e = jnp.dot(xyz_ref[...], wd1_ref[...], preferred_element_type=jnp.float32)
    e_ref[...] = e
    t_ref[:, 0:DM] = jnp.dot(x, wk_ref[...], preferred_element_type=jnp.float32)
    t_ref[:, DM:2 * DM] = jnp.dot(x, wv_ref[...],
                                  preferred_element_type=jnp.float32)
    t_ref[:, 2 * DM:TD] = e


def _pre(feats, xyz32, wf, bf, wq, wk, wv, wd1):
    n, c = feats.shape
    m = min(512, n)
    return pl.pallas_call(
        _pre_body,
        grid=(n // m,),
        in_specs=[
            pl.BlockSpec((m, c), lambda i: (i, 0)),
            pl.BlockSpec((m, XP), lambda i: (i, 0)),
            pl.BlockSpec((c, DM), lambda i: (0, 0)),
            pl.BlockSpec((1, DM), lambda i: (0, 0)),
            pl.BlockSpec((DM, DM), lambda i: (0, 0)),
            pl.BlockSpec((DM, DM), lambda i: (0, 0)),
            pl.BlockSpec((DM, DM), lambda i: (0, 0)),
            pl.BlockSpec((XP, DM), lambda i: (0, 0)),
        ],
        out_specs=[
            pl.BlockSpec((m, DM), lambda i: (i, 0)),
            pl.BlockSpec((m, DM), lambda i: (i, 0)),
            pl.BlockSpec((m, TD), lambda i: (i, 0)),
        ],
        out_shape=[
            jax.ShapeDtypeStruct((n, DM), jnp.float32),
            jax.ShapeDtypeStruct((n, DM), jnp.float32),
            jax.ShapeDtypeStruct((n, TD), jnp.float32),
        ],
        compiler_params=pltpu.CompilerParams(dimension_semantics=("parallel",)),
    )(feats, xyz32, wf, bf, wq, wk, wv, wd1)


def _make_sc_gather(n, b):
    bpw = b // NW
    ch = min(bpw, 64)
    nloop = bpw // ch
    mesh = plsc.VectorSubcoreMesh(core_axis_name="c", subcore_axis_name="s",
                                  num_cores=NC, num_subcores=NS)

    def body(tab_ref, idx_ref, out_ref, idx_v, rows_v, sem):
        wid = jax.lax.axis_index("s") * NC + jax.lax.axis_index("c")
        base = wid * bpw

        def step(i, carry):
            off = base + i * ch
            pltpu.sync_copy(idx_ref.at[pl.ds(off, ch)], idx_v)
            pltpu.async_copy(tab_ref.at[idx_v], rows_v, sem).wait()
            pltpu.sync_copy(rows_v, out_ref.at[pl.ds(off, ch)])
            return carry

        jax.lax.fori_loop(0, nloop, step, 0)

    return pl.kernel(
        body,
        out_type=jax.ShapeDtypeStruct((b, TD), jnp.float32),
        mesh=mesh,
        scratch_types=[
            pltpu.VMEM((ch,), jnp.int32),
            pltpu.VMEM((ch, TD), jnp.float32),
            pltpu.SemaphoreType.DMA,
        ],
    )


def _sc_gather(table, flat_idx):
    n = table.shape[0]
    b = flat_idx.shape[0]
    return _make_sc_gather(n, b)(table, flat_idx)


def _attn_body(g_ref, q_ref, eq_ref, pre_ref, bd1, wd2, bd2,
               wg1, bg1, wg2, bg2, wo, bo, o_ref, *, m):
    kk = g_ref[:, 0:DM]
    vv = g_ref[:, DM:2 * DM]
    ee = g_ref[:, 2 * DM:TD]
    q3 = jnp.broadcast_to(q_ref[...][:, None, :], (m, K, DM))
    e3 = jnp.broadcast_to(eq_ref[...][:, None, :], (m, K, DM))
    p1 = jnp.maximum(e3.reshape(m * K, DM) - ee + bd1[...], 0.0)
    pos = jnp.dot(p1, wd2[...], preferred_element_type=jnp.float32) + bd2[...]
    t = q3.reshape(m * K, DM) - kk + pos
    a1 = jnp.maximum(
        jnp.dot(t, wg1[...], preferred_element_type=jnp.float32) + bg1[...],
        0.0)
    lg = (jnp.dot(a1, wg2[...], preferred_element_type=jnp.float32)
          + bg2[...]) * SM_SCALE
    lg3 = lg.reshape(m, K, DM)
    mx = jnp.max(lg3, axis=1, keepdims=True)
    e = jnp.exp(lg3 - mx)
    s = jnp.sum(e, axis=1, keepdims=True)
    vp = (vv + pos).reshape(m, K, DM)
    r = jnp.sum((e / s) * vp, axis=1)
    o_ref[...] = (jnp.dot(r, wo[...], preferred_element_type=jnp.float32)
                  + bo[...] + pre_ref[...])


def _attn(g, q, eq, pre, bd1, wd2, bd2, wg1, bg1, wg2, bg2, wo, bo):
    n, c = pre.shape
    m = min(128, n)
    return pl.pallas_call(
        functools.partial(_attn_body, m=m),
        grid=(n // m,),
        in_specs=[
            pl.BlockSpec((m * K, TD), lambda i: (i, 0)),
            pl.BlockSpec((m, DM), lambda i: (i, 0)),
            pl.BlockSpec((m, DM), lambda i: (i, 0)),
            pl.BlockSpec((m, c), lambda i: (i, 0)),
            pl.BlockSpec((1, DM), lambda i: (0, 0)),
            pl.BlockSpec((DM, DM), lambda i: (0, 0)),
            pl.BlockSpec((1, DM), lambda i: (0, 0)),
            pl.BlockSpec((DM, DM), lambda i: (0, 0)),
            pl.BlockSpec((1, DM), lambda i: (0, 0)),
            pl.BlockSpec((DM, DM), lambda i: (0, 0)),
            pl.BlockSpec((1, DM), lambda i: (0, 0)),
            pl.BlockSpec((DM, c), lambda i: (0, 0)),
            pl.BlockSpec((1, c), lambda i: (0, 0)),
        ],
        out_specs=pl.BlockSpec((m, c), lambda i: (i, 0)),
        out_shape=jax.ShapeDtypeStruct((n, c), jnp.float32),
        compiler_params=pltpu.CompilerParams(dimension_semantics=("parallel",)),
    )(g, q, eq, pre, bd1, wd2, bd2, wg1, bg1, wg2, bg2, wo, bo)


def _tu_body(xf_ref, xct_ref, pc_ref, ff_ref, w1, b1, g1, t1,
             w2, b2, g2, t2, o_ref):
    f2 = jnp.maximum(
        (jnp.dot(ff_ref[...], w2[...], preferred_element_type=jnp.float32)
         + b2[...]) * (g2[...] * BN_INV) + t2[...], 0.0)
    f1 = jnp.maximum(
        (jnp.dot(pc_ref[...], w1[...], preferred_element_type=jnp.float32)
         + b1[...]) * (g1[...] * BN_INV) + t1[...], 0.0)
    qb = xf_ref[...]
    xt = xct_ref[...]
    d = (jnp.sum(qb * qb, axis=1, keepdims=True)
         + jnp.sum(xt * xt, axis=0, keepdims=True)
         - 2.0 * jnp.dot(qb, xt, preferred_element_type=jnp.float32,
                         precision=jax.lax.Precision.HIGHEST))
    d = jnp.maximum(d, 0.0)
    cols = jax.lax.broadcasted_iota(jnp.int32, d.shape, 1)
    recs, sels = [], []
    for _ in range(3):
        mn = jnp.min(d, axis=1, keepdims=True)
        idx = jnp.min(jnp.where(d <= mn, cols, BIGI), axis=1, keepdims=True)
        recs.append(1.0 / (mn + 1e-8))
        sels.append(cols == idx)
        d = jnp.where(cols == idx, BIGF, d)
    norm = recs[0] + recs[1] + recs[2]
    w = (jnp.where(sels[0], recs[0] / norm, 0.0)
         + jnp.where(sels[1], recs[1] / norm, 0.0)
         + jnp.where(sels[2], recs[2] / norm, 0.0))
    interp = jnp.dot(w, f1, preferred_element_type=jnp.float32)
    o_ref[...] = interp + f2


def _tu(p, xyzf8, xyzc8t, points_coarse, feats_fine):
    nf, c = feats_fine.shape
    s = points_coarse.shape[0]
    mf = min(256, nf)
    c2 = points_coarse.shape[1]
    r2 = lambda a: a.reshape(1, -1)
    return pl.pallas_call(
        _tu_body,
        grid=(nf // mf,),
        in_specs=[
            pl.BlockSpec((mf, 8), lambda i: (i, 0)),
            pl.BlockSpec((8, s), lambda i: (0, 0)),
            pl.BlockSpec((s, c2), lambda i: (0, 0)),
            pl.BlockSpec((mf, c), lambda i: (i, 0)),
        ] + [pl.BlockSpec(bs, lambda i: (0, 0)) for bs in
             [(c2, c), (1, c), (1, c), (1, c),
              (c, c), (1, c), (1, c), (1, c)]],
        out_specs=pl.BlockSpec((mf, c), lambda i: (i, 0)),
        out_shape=jax.ShapeDtypeStruct((nf, c), jnp.float32),
        compiler_params=pltpu.CompilerParams(dimension_semantics=("parallel",)),
    )(xyzf8, xyzc8t, points_coarse, feats_fine,
      p["fc1"]["w"], r2(p["fc1"]["b"]), r2(p["bn1"]["gamma"]), r2(p["bn1"]["beta"]),
      p["fc2"]["w"], r2(p["fc2"]["b"]), r2(p["bn2"]["gamma"]), r2(p["bn2"]["beta"]))


def _mlp_body(x_ref, w1, b1, w2, b2, w3, b3, o_ref):
    h = jnp.maximum(jnp.dot(x_ref[...], w1[...],
                            preferred_element_type=jnp.float32) + b1[...], 0.0)
    h = jnp.maximum(jnp.dot(h, w2[...],
                            preferred_element_type=jnp.float32) + b2[...], 0.0)
    o_ref[...] = jnp.dot(h, w3[...],
                         preferred_element_type=jnp.float32) + b3[...]


def _mlp(x, f):
    n, c = x.shape
    r2 = lambda a: a.reshape(1, -1)
    return pl.pallas_call(
        _mlp_body,
        out_shape=jax.ShapeDtypeStruct((n, c), jnp.float32),
    )(x, f["l1"]["w"], r2(f["l1"]["b"]), f["l2"]["w"], r2(f["l2"]["b"]),
      f["l3"]["w"], r2(f["l3"]["b"]))


def _tb(p, idx, xyz32, feats):
    r2 = lambda a: a.reshape(1, -1)
    wd1 = jnp.pad(p["delta1"]["w"], ((0, XP - 3), (0, 0)))
    q, e, table = _pre(feats, xyz32, p["fc1"]["w"], r2(p["fc1"]["b"]),
                       p["wq"]["w"], p["wk"]["w"], p["wv"]["w"], wd1)
    g = _sc_gather(table, idx.reshape(-1))
    return _attn(g, q, e, feats,
                 r2(p["delta1"]["b"]), p["delta2"]["w"], r2(p["delta2"]["b"]),
                 p["gamma1"]["w"], r2(p["gamma1"]["b"]),
                 p["gamma2"]["w"], r2(p["gamma2"]["b"]),
                 p["fc2"]["w"], r2(p["fc2"]["b"]))


def kernel(points, xyz0, xyz1, xyz2, xyz3, xyz4, feats0, feats1, feats2,
           feats3, params):
    xyzs = [x[0] for x in (xyz0, xyz1, xyz2, xyz3, xyz4)]
    featss = [f[0] for f in (feats0, feats1, feats2, feats3)]
    pad8 = [jnp.pad(x, ((0, 0), (0, 8 - 3))) for x in xyzs]
    pad8t = [x.T for x in pad8]
    pad32 = [jnp.pad(x, ((0, 0), (0, XP - 3))) for x in xyzs]

    knn_idx = [_knn(pad8[l], pad8t[l], K) for l in range(5)]

    h = _mlp(points[0], params["fc"])
    lvl = 4
    pts = _tb(params["transformer"], knn_idx[lvl], pad32[lvl], h)
    for i in range(4):
        fine = 3 - i
        pts = _tu(params["tu"][i], pad8[fine], pad8t[lvl],
                  pts, featss[fine])
        lvl = fine
        pts = _tb(params["tr"][i], knn_idx[lvl], pad32[lvl], pts)
    return pts[None]


# speedup vs baseline: 1888.8675x; 1.0302x over previous
import functools

import jax
import jax.numpy as jnp
import numpy as np
from jax.experimental import pallas as pl
from jax.experimental.pallas import tpu as pltpu
from jax.experimental.pallas import tpu_sc as plsc

K = 16
DM = 256
XP = 32
TD = 3 * DM
NC, NS = 2, 16
NW = NC * NS
BIGF = np.float32(1e30)
BIGI = np.int32(1 << 30)
BN_INV = np.float32(1.0 / np.sqrt(1.0 + 1e-5))
SM_SCALE = np.float32(1.0 / 16.0)


def _knn_body(q_ref, xt_ref, o_ref, *, k):
    qb = q_ref[...]
    xt = xt_ref[...]
    d = (jnp.sum(qb * qb, axis=1, keepdims=True)
         + jnp.sum(xt * xt, axis=0, keepdims=True)
         - 2.0 * jnp.dot(qb, xt, preferred_element_type=jnp.float32,
                         precision=jax.lax.Precision.HIGHEST))
    cols = jax.lax.broadcasted_iota(jnp.int32, d.shape, 1).astype(jnp.float32)
    outs = []
    for _ in range(k):
        m = jnp.min(d, axis=1, keepdims=True)
        idx = jnp.min(jnp.where(d <= m, cols, BIGF), axis=1, keepdims=True)
        outs.append(idx)
        d = jnp.where(cols == idx, BIGF, d)
    o_ref[...] = jnp.concatenate(outs, axis=1).astype(jnp.int32)


def _knn(xyz8, xyz8t, k):
    n = xyz8.shape[0]
    m = min(256, n)
    return pl.pallas_call(
        functools.partial(_knn_body, k=k),
        grid=(n // m,),
        in_specs=[
            pl.BlockSpec((m, 8), lambda i: (i, 0)),
            pl.BlockSpec((8, n), lambda i: (0, 0)),
        ],
        out_specs=pl.BlockSpec((m, k), lambda i: (i, 0)),
        out_shape=jax.ShapeDtypeStruct((n, k), jnp.int32),
        compiler_params=pltpu.CompilerParams(dimension_semantics=("parallel",)),
    )(xyz8, xyz8t)


def _pre_body(f_ref, xyz_ref, wf_ref, bf_ref, wq_ref, wk_ref, wv_ref, wd1_ref,
              q_ref, e_ref, t_ref):
    x = jnp.dot(f_ref[...], wf_ref[...],
                preferred_element_type=jnp.float32) + bf_ref[...]
    q_ref[...] = jnp.dot(x, wq_ref[...], preferred_element_type=jnp.float32)
    e = jnp.dot(xyz_ref[...], wd1_ref[...], preferred_element_type=jnp.float32)
    e_ref[...] = e
    t_ref[:, 0:DM] = jnp.dot(x, wk_ref[...], preferred_element_type=jnp.float32)
    t_ref[:, DM:2 * DM] = jnp.dot(x, wv_ref[...],
                                  preferred_element_type=jnp.float32)
    t_ref[:, 2 * DM:TD] = e


def _pre(feats, xyz32, wf, bf, wq, wk, wv, wd1):
    n, c = feats.shape
    m = min(512, n)
    return pl.pallas_call(
        _pre_body,
        grid=(n // m,),
        in_specs=[
            pl.BlockSpec((m, c), lambda i: (i, 0)),
            pl.BlockSpec((m, XP), lambda i: (i, 0)),
            pl.BlockSpec((c, DM), lambda i: (0, 0)),
            pl.BlockSpec((1, DM), lambda i: (0, 0)),
            pl.BlockSpec((DM, DM), lambda i: (0, 0)),
            pl.BlockSpec((DM, DM), lambda i: (0, 0)),
            pl.BlockSpec((DM, DM), lambda i: (0, 0)),
            pl.BlockSpec((XP, DM), lambda i: (0, 0)),
        ],
        out_specs=[
            pl.BlockSpec((m, DM), lambda i: (i, 0)),
            pl.BlockSpec((m, DM), lambda i: (i, 0)),
            pl.BlockSpec((m, TD), lambda i: (i, 0)),
        ],
        out_shape=[
            jax.ShapeDtypeStruct((n, DM), jnp.float32),
            jax.ShapeDtypeStruct((n, DM), jnp.float32),
            jax.ShapeDtypeStruct((n, TD), jnp.float32),
        ],
        compiler_params=pltpu.CompilerParams(dimension_semantics=("parallel",)),
    )(feats, xyz32, wf, bf, wq, wk, wv, wd1)


def _make_sc_gather(n, b):
    bpw = b // NW
    ch = min(bpw, 64)
    nloop = bpw // ch
    mesh = plsc.VectorSubcoreMesh(core_axis_name="c", subcore_axis_name="s",
                                  num_cores=NC, num_subcores=NS)

    def body(tab_ref, idx_ref, out_ref, idx0, idx1, r0, r1, semg, semo):
        wid = jax.lax.axis_index("s") * NC + jax.lax.axis_index("c")
        base = wid * bpw
        idx_b = (idx0, idx1)
        rows_b = (r0, r1)

        if nloop == 1:
            pltpu.sync_copy(idx_ref.at[pl.ds(base, ch)], idx0)
            pltpu.async_copy(tab_ref.at[idx0], r0, semg).wait()
            pltpu.sync_copy(r0, out_ref.at[pl.ds(base, ch)])
            return

        @pl.loop(0, nloop, step=2)
        def _outer(i0):
            for bsel in range(2):
                i = i0 + bsel
                off = base + i * ch

                @pl.when(i >= 2)
                def _drain():
                    pltpu.make_async_copy(
                        rows_b[bsel], out_ref.at[pl.ds(base, ch)], semo).wait()

                pltpu.sync_copy(idx_ref.at[pl.ds(off, ch)], idx_b[bsel])
                pltpu.async_copy(tab_ref.at[idx_b[bsel]], rows_b[bsel],
                                 semg).wait()
                pltpu.async_copy(rows_b[bsel], out_ref.at[pl.ds(off, ch)],
                                 semo)

        for bsel in range(2):
            pltpu.make_async_copy(
                rows_b[bsel], out_ref.at[pl.ds(base, ch)], semo).wait()

    return pl.kernel(
        body,
        out_type=jax.ShapeDtypeStruct((b, TD), jnp.float32),
        mesh=mesh,
        scratch_types=[
            pltpu.VMEM((ch,), jnp.int32),
            pltpu.VMEM((ch,), jnp.int32),
            pltpu.VMEM((ch, TD), jnp.float32),
            pltpu.VMEM((ch, TD), jnp.float32),
            pltpu.SemaphoreType.DMA,
            pltpu.SemaphoreType.DMA,
        ],
    )


def _sc_gather(table, flat_idx):
    n = table.shape[0]
    b = flat_idx.shape[0]
    return _make_sc_gather(n, b)(table, flat_idx)


def _attn_body(g_ref, q_ref, eq_ref, pre_ref, bd1, wd2, bd2,
               wg1, bg1, wg2, bg2, wo, bo, o_ref, *, m):
    kk = g_ref[:, 0:DM]
    vv = g_ref[:, DM:2 * DM]
    ee = g_ref[:, 2 * DM:TD]
    q3 = jnp.broadcast_to(q_ref[...][:, None, :], (m, K, DM))
    e3 = jnp.broadcast_to(eq_ref[...][:, None, :], (m, K, DM))
    p1 = jnp.maximum(e3.reshape(m * K, DM) - ee + bd1[...], 0.0)
    pos = jnp.dot(p1, wd2[...], preferred_element_type=jnp.float32) + bd2[...]
    t = q3.reshape(m * K, DM) - kk + pos
    a1 = jnp.maximum(
        jnp.dot(t, wg1[...], preferred_element_type=jnp.float32) + bg1[...],
        0.0)
    lg = (jnp.dot(a1, wg2[...], preferred_element_type=jnp.float32)
          + bg2[...]) * SM_SCALE
    lg3 = lg.reshape(m, K, DM)
    mx = jnp.max(lg3, axis=1, keepdims=True)
    e = jnp.exp(lg3 - mx)
    s = jnp.sum(e, axis=1, keepdims=True)
    vp = (vv + pos).reshape(m, K, DM)
    r = jnp.sum((e / s) * vp, axis=1)
    o_ref[...] = (jnp.dot(r, wo[...], preferred_element_type=jnp.float32)
                  + bo[...] + pre_ref[...])


def _attn(g, q, eq, pre, bd1, wd2, bd2, wg1, bg1, wg2, bg2, wo, bo):
    n, c = pre.shape
    m = min(128, n)
    return pl.pallas_call(
        functools.partial(_attn_body, m=m),
        grid=(n // m,),
        in_specs=[
            pl.BlockSpec((m * K, TD), lambda i: (i, 0)),
            pl.BlockSpec((m, DM), lambda i: (i, 0)),
            pl.BlockSpec((m, DM), lambda i: (i, 0)),
            pl.BlockSpec((m, c), lambda i: (i, 0)),
            pl.BlockSpec((1, DM), lambda i: (0, 0)),
            pl.BlockSpec((DM, DM), lambda i: (0, 0)),
            pl.BlockSpec((1, DM), lambda i: (0, 0)),
            pl.BlockSpec((DM, DM), lambda i: (0, 0)),
            pl.BlockSpec((1, DM), lambda i: (0, 0)),
            pl.BlockSpec((DM, DM), lambda i: (0, 0)),
            pl.BlockSpec((1, DM), lambda i: (0, 0)),
            pl.BlockSpec((DM, c), lambda i: (0, 0)),
            pl.BlockSpec((1, c), lambda i: (0, 0)),
        ],
        out_specs=pl.BlockSpec((m, c), lambda i: (i, 0)),
        out_shape=jax.ShapeDtypeStruct((n, c), jnp.float32),
        compiler_params=pltpu.CompilerParams(dimension_semantics=("parallel",)),
    )(g, q, eq, pre, bd1, wd2, bd2, wg1, bg1, wg2, bg2, wo, bo)


def _tu_body(xf_ref, xct_ref, pc_ref, ff_ref, w1, b1, g1, t1,
             w2, b2, g2, t2, o_ref):
    f2 = jnp.maximum(
        (jnp.dot(ff_ref[...], w2[...], preferred_element_type=jnp.float32)
         + b2[...]) * (g2[...] * BN_INV) + t2[...], 0.0)
    f1 = jnp.maximum(
        (jnp.dot(pc_ref[...], w1[...], preferred_element_type=jnp.float32)
         + b1[...]) * (g1[...] * BN_INV) + t1[...], 0.0)
    qb = xf_ref[...]
    xt = xct_ref[...]
    d = (jnp.sum(qb * qb, axis=1, keepdims=True)
         + jnp.sum(xt * xt, axis=0, keepdims=True)
         - 2.0 * jnp.dot(qb, xt, preferred_element_type=jnp.float32,
                         precision=jax.lax.Precision.HIGHEST))
    d = jnp.maximum(d, 0.0)
    cols = jax.lax.broadcasted_iota(jnp.int32, d.shape, 1)
    recs, sels = [], []
    for _ in range(3):
        mn = jnp.min(d, axis=1, keepdims=True)
        idx = jnp.min(jnp.where(d <= mn, cols, BIGI), axis=1, keepdims=True)
        recs.append(1.0 / (mn + 1e-8))
        sels.append(cols == idx)
        d = jnp.where(cols == idx, BIGF, d)
    norm = recs[0] + recs[1] + recs[2]
    w = (jnp.where(sels[0], recs[0] / norm, 0.0)
         + jnp.where(sels[1], recs[1] / norm, 0.0)
         + jnp.where(sels[2], recs[2] / norm, 0.0))
    interp = jnp.dot(w, f1, preferred_element_type=jnp.float32)
    o_ref[...] = interp + f2


def _tu(p, xyzf8, xyzc8t, points_coarse, feats_fine):
    nf, c = feats_fine.shape
    s = points_coarse.shape[0]
    mf = min(256, nf)
    c2 = points_coarse.shape[1]
    r2 = lambda a: a.reshape(1, -1)
    return pl.pallas_call(
        _tu_body,
        grid=(nf // mf,),
        in_specs=[
            pl.BlockSpec((mf, 8), lambda i: (i, 0)),
            pl.BlockSpec((8, s), lambda i: (0, 0)),
            pl.BlockSpec((s, c2), lambda i: (0, 0)),
            pl.BlockSpec((mf, c), lambda i: (i, 0)),
        ] + [pl.BlockSpec(bs, lambda i: (0, 0)) for bs in
             [(c2, c), (1, c), (1, c), (1, c),
              (c, c), (1, c), (1, c), (1, c)]],
        out_specs=pl.BlockSpec((mf, c), lambda i: (i, 0)),
        out_shape=jax.ShapeDtypeStruct((nf, c), jnp.float32),
        compiler_params=pltpu.CompilerParams(dimension_semantics=("parallel",)),
    )(xyzf8, xyzc8t, points_coarse, feats_fine,
      p["fc1"]["w"], r2(p["fc1"]["b"]), r2(p["bn1"]["gamma"]), r2(p["bn1"]["beta"]),
      p["fc2"]["w"], r2(p["fc2"]["b"]), r2(p["bn2"]["gamma"]), r2(p["bn2"]["beta"]))


def _mlp_body(x_ref, w1, b1, w2, b2, w3, b3, o_ref):
    h = jnp.maximum(jnp.dot(x_ref[...], w1[...],
                            preferred_element_type=jnp.float32) + b1[...], 0.0)
    h = jnp.maximum(jnp.dot(h, w2[...],
                            preferred_element_type=jnp.float32) + b2[...], 0.0)
    o_ref[...] = jnp.dot(h, w3[...],
                         preferred_element_type=jnp.float32) + b3[...]


def _mlp(x, f):
    n, c = x.shape
    r2 = lambda a: a.reshape(1, -1)
    return pl.pallas_call(
        _mlp_body,
        out_shape=jax.ShapeDtypeStruct((n, c), jnp.float32),
    )(x, f["l1"]["w"], r2(f["l1"]["b"]), f["l2"]["w"], r2(f["l2"]["b"]),
      f["l3"]["w"], r2(f["l3"]["b"]))


def _tb(p, idx, xyz32, feats):
    r2 = lambda a: a.reshape(1, -1)
    wd1 = jnp.pad(p["delta1"]["w"], ((0, XP - 3), (0, 0)))
    q, e, table = _pre(feats, xyz32, p["fc1"]["w"], r2(p["fc1"]["b"]),
                       p["wq"]["w"], p["wk"]["w"], p["wv"]["w"], wd1)
    g = _sc_gather(table, idx.reshape(-1))
    return _attn(g, q, e, feats,
                 r2(p["delta1"]["b"]), p["delta2"]["w"], r2(p["delta2"]["b"]),
                 p["gamma1"]["w"], r2(p["gamma1"]["b"]),
                 p["gamma2"]["w"], r2(p["gamma2"]["b"]),
                 p["fc2"]["w"], r2(p["fc2"]["b"]))


def kernel(points, xyz0, xyz1, xyz2, xyz3, xyz4, feats0, feats1, feats2,
           feats3, params):
    xyzs = [x[0] for x in (xyz0, xyz1, xyz2, xyz3, xyz4)]
    featss = [f[0] for f in (feats0, feats1, feats2, feats3)]
    pad8 = [jnp.pad(x, ((0, 0), (0, 8 - 3))) for x in xyzs]
    pad8t = [x.T for x in pad8]
    pad32 = [jnp.pad(x, ((0, 0), (0, XP - 3))) for x in xyzs]

    knn_idx = [_knn(pad8[l], pad8t[l], K) for l in range(5)]

    h = _mlp(points[0], params["fc"])
    lvl = 4
    pts = _tb(params["transformer"], knn_idx[lvl], pad32[lvl], h)
    for i in range(4):
        fine = 3 - i
        pts = _tu(params["tu"][i], pad8[fine], pad8t[lvl],
                  pts, featss[fine])
        lvl = fine
        pts = _tb(params["tr"][i], knn_idx[lvl], pad32[lvl], pts)
    return pts[None]


# speedup vs baseline: 1981.7691x; 1.0492x over previous
import functools

import jax
import jax.numpy as jnp
import numpy as np
from jax.experimental import pallas as pl
from jax.experimental.pallas import tpu as pltpu
from jax.experimental.pallas import tpu_sc as plsc

K = 16
DM = 256
XP = 32
TD = 3 * DM
NC, NS = 2, 16
NW = NC * NS
BIGF = np.float32(1e30)
BIGI = np.int32(1 << 30)
BN_INV = np.float32(1.0 / np.sqrt(1.0 + 1e-5))
SM_SCALE = np.float32(1.0 / 16.0)


def _sqdist(qb, xt):
    return (jnp.sum(qb * qb, axis=1, keepdims=True)
            + jnp.sum(xt * xt, axis=0, keepdims=True)
            - 2.0 * jnp.dot(qb, xt, preferred_element_type=jnp.float32,
                            precision=jax.lax.Precision.HIGHEST))


def _topk_cols(d, k):
    cols = jax.lax.broadcasted_iota(jnp.int32, d.shape, 1).astype(jnp.float32)
    outs = []
    for _ in range(k):
        m = jnp.min(d, axis=1, keepdims=True)
        idx = jnp.min(jnp.where(d <= m, cols, BIGF), axis=1, keepdims=True)
        outs.append(idx)
        d = jnp.where(cols == idx, BIGF, d)
    return outs


def _attn_core(qx, eqx, kk, vv, ee, bd1, wd2, bd2, wg1, bg1, wg2, bg2, m):
    p1 = jnp.maximum(eqx - ee + bd1, 0.0)
    pos = jnp.dot(p1, wd2, preferred_element_type=jnp.float32) + bd2
    t = qx - kk + pos
    a1 = jnp.maximum(jnp.dot(t, wg1, preferred_element_type=jnp.float32)
                     + bg1, 0.0)
    lg = (jnp.dot(a1, wg2, preferred_element_type=jnp.float32)
          + bg2) * SM_SCALE
    lg3 = lg.reshape(m, K, DM)
    mx = jnp.max(lg3, axis=1, keepdims=True)
    e = jnp.exp(lg3 - mx)
    s = jnp.sum(e, axis=1, keepdims=True)
    vp = (vv + pos).reshape(m, K, DM)
    return jnp.sum((e / s) * vp, axis=1)


def _tu_core(xf8, xct, pc, ff, w1, b1, g1, t1, w2, b2, g2, t2):
    f2 = jnp.maximum(
        (jnp.dot(ff, w2, preferred_element_type=jnp.float32)
         + b2) * (g2 * BN_INV) + t2, 0.0)
    f1 = jnp.maximum(
        (jnp.dot(pc, w1, preferred_element_type=jnp.float32)
         + b1) * (g1 * BN_INV) + t1, 0.0)
    d = jnp.maximum(_sqdist(xf8, xct), 0.0)
    cols = jax.lax.broadcasted_iota(jnp.int32, d.shape, 1).astype(jnp.float32)
    recs, sels = [], []
    for _ in range(3):
        mn = jnp.min(d, axis=1, keepdims=True)
        idx = jnp.min(jnp.where(d <= mn, cols, BIGF), axis=1, keepdims=True)
        recs.append(1.0 / (mn + 1e-8))
        sels.append(cols == idx)
        d = jnp.where(cols == idx, BIGF, d)
    norm = recs[0] + recs[1] + recs[2]
    w = (jnp.where(sels[0], recs[0] / norm, 0.0)
         + jnp.where(sels[1], recs[1] / norm, 0.0)
         + jnp.where(sels[2], recs[2] / norm, 0.0))
    return jnp.dot(w, f1, preferred_element_type=jnp.float32) + f2


def _knn_body(q_ref, xt_ref, o_ref, *, k):
    d = _sqdist(q_ref[...], xt_ref[...])
    outs = _topk_cols(d, k)
    o_ref[...] = jnp.concatenate(outs, axis=1).astype(jnp.int32)


def _knn(xyz8, xyz8t, k):
    n = xyz8.shape[0]
    m = min(256, n)
    return pl.pallas_call(
        functools.partial(_knn_body, k=k),
        grid=(n // m,),
        in_specs=[
            pl.BlockSpec((m, 8), lambda i: (i, 0)),
            pl.BlockSpec((8, n), lambda i: (0, 0)),
        ],
        out_specs=pl.BlockSpec((m, k), lambda i: (i, 0)),
        out_shape=jax.ShapeDtypeStruct((n, k), jnp.int32),
        compiler_params=pltpu.CompilerParams(dimension_semantics=("parallel",)),
    )(xyz8, xyz8t)


def _pre_body(f_ref, xyz_ref, wf_ref, bf_ref, wq_ref, wk_ref, wv_ref, wd1_ref,
              q_ref, e_ref, t_ref):
    x = jnp.dot(f_ref[...], wf_ref[...],
                preferred_element_type=jnp.float32) + bf_ref[...]
    q_ref[...] = jnp.dot(x, wq_ref[...], preferred_element_type=jnp.float32)
    e = jnp.dot(xyz_ref[...], wd1_ref[...], preferred_element_type=jnp.float32)
    e_ref[...] = e
    t_ref[:, 0:DM] = jnp.dot(x, wk_ref[...], preferred_element_type=jnp.float32)
    t_ref[:, DM:2 * DM] = jnp.dot(x, wv_ref[...],
                                  preferred_element_type=jnp.float32)
    t_ref[:, 2 * DM:TD] = e


def _pre(feats, xyz32, wf, bf, wq, wk, wv, wd1):
    n, c = feats.shape
    m = min(512, n)
    return pl.pallas_call(
        _pre_body,
        grid=(n // m,),
        in_specs=[
            pl.BlockSpec((m, c), lambda i: (i, 0)),
            pl.BlockSpec((m, XP), lambda i: (i, 0)),
            pl.BlockSpec((c, DM), lambda i: (0, 0)),
            pl.BlockSpec((1, DM), lambda i: (0, 0)),
            pl.BlockSpec((DM, DM), lambda i: (0, 0)),
            pl.BlockSpec((DM, DM), lambda i: (0, 0)),
            pl.BlockSpec((DM, DM), lambda i: (0, 0)),
            pl.BlockSpec((XP, DM), lambda i: (0, 0)),
        ],
        out_specs=[
            pl.BlockSpec((m, DM), lambda i: (i, 0)),
            pl.BlockSpec((m, DM), lambda i: (i, 0)),
            pl.BlockSpec((m, TD), lambda i: (i, 0)),
        ],
        out_shape=[
            jax.ShapeDtypeStruct((n, DM), jnp.float32),
            jax.ShapeDtypeStruct((n, DM), jnp.float32),
            jax.ShapeDtypeStruct((n, TD), jnp.float32),
        ],
        compiler_params=pltpu.CompilerParams(dimension_semantics=("parallel",)),
    )(feats, xyz32, wf, bf, wq, wk, wv, wd1)


def _make_sc_gather(n, b):
    bpw = b // NW
    ch = min(bpw, 64)
    nloop = bpw // ch
    mesh = plsc.VectorSubcoreMesh(core_axis_name="c", subcore_axis_name="s",
                                  num_cores=NC, num_subcores=NS)

    def body(tab_ref, idx_ref, out_ref, idx0, idx1, r0, r1, semg, semo):
        wid = jax.lax.axis_index("s") * NC + jax.lax.axis_index("c")
        base = wid * bpw
        idx_b = (idx0, idx1)
        rows_b = (r0, r1)

        if nloop == 1:
            pltpu.sync_copy(idx_ref.at[pl.ds(base, ch)], idx0)
            pltpu.async_copy(tab_ref.at[idx0], r0, semg).wait()
            pltpu.sync_copy(r0, out_ref.at[pl.ds(base, ch)])
            return

        @pl.loop(0, nloop, step=2)
        def _outer(i0):
            for bsel in range(2):
                i = i0 + bsel
                off = base + i * ch

                @pl.when(i >= 2)
                def _drain():
                    pltpu.make_async_copy(
                        rows_b[bsel], out_ref.at[pl.ds(base, ch)], semo).wait()

                pltpu.sync_copy(idx_ref.at[pl.ds(off, ch)], idx_b[bsel])
                pltpu.async_copy(tab_ref.at[idx_b[bsel]], rows_b[bsel],
                                 semg).wait()
                pltpu.async_copy(rows_b[bsel], out_ref.at[pl.ds(off, ch)],
                                 semo)

        for bsel in range(2):
            pltpu.make_async_copy(
                rows_b[bsel], out_ref.at[pl.ds(base, ch)], semo).wait()

    return pl.kernel(
        body,
        out_type=jax.ShapeDtypeStruct((b, TD), jnp.float32),
        mesh=mesh,
        scratch_types=[
            pltpu.VMEM((ch,), jnp.int32),
            pltpu.VMEM((ch,), jnp.int32),
            pltpu.VMEM((ch, TD), jnp.float32),
            pltpu.VMEM((ch, TD), jnp.float32),
            pltpu.SemaphoreType.DMA,
            pltpu.SemaphoreType.DMA,
        ],
    )


def _sc_gather(table, flat_idx):
    n = table.shape[0]
    b = flat_idx.shape[0]
    return _make_sc_gather(n, b)(table, flat_idx)


def _attn_body(g_ref, q_ref, eq_ref, pre_ref, bd1, wd2, bd2,
               wg1, bg1, wg2, bg2, wo, bo, o_ref, *, m):
    kk = g_ref[:, 0:DM]
    vv = g_ref[:, DM:2 * DM]
    ee = g_ref[:, 2 * DM:TD]
    qx = jnp.broadcast_to(q_ref[...][:, None, :], (m, K, DM)).reshape(m * K, DM)
    eqx = jnp.broadcast_to(eq_ref[...][:, None, :],
                           (m, K, DM)).reshape(m * K, DM)
    r = _attn_core(qx, eqx, kk, vv, ee, bd1[...], wd2[...], bd2[...],
                   wg1[...], bg1[...], wg2[...], bg2[...], m)
    o_ref[...] = (jnp.dot(r, wo[...], preferred_element_type=jnp.float32)
                  + bo[...] + pre_ref[...])


def _attn(g, q, eq, pre, bd1, wd2, bd2, wg1, bg1, wg2, bg2, wo, bo):
    n, c = pre.shape
    m = min(128, n)
    return pl.pallas_call(
        functools.partial(_attn_body, m=m),
        grid=(n // m,),
        in_specs=[
            pl.BlockSpec((m * K, TD), lambda i: (i, 0)),
            pl.BlockSpec((m, DM), lambda i: (i, 0)),
            pl.BlockSpec((m, DM), lambda i: (i, 0)),
            pl.BlockSpec((m, c), lambda i: (i, 0)),
            pl.BlockSpec((1, DM), lambda i: (0, 0)),
            pl.BlockSpec((DM, DM), lambda i: (0, 0)),
            pl.BlockSpec((1, DM), lambda i: (0, 0)),
            pl.BlockSpec((DM, DM), lambda i: (0, 0)),
            pl.BlockSpec((1, DM), lambda i: (0, 0)),
            pl.BlockSpec((DM, DM), lambda i: (0, 0)),
            pl.BlockSpec((1, DM), lambda i: (0, 0)),
            pl.BlockSpec((DM, c), lambda i: (0, 0)),
            pl.BlockSpec((1, c), lambda i: (0, 0)),
        ],
        out_specs=pl.BlockSpec((m, c), lambda i: (i, 0)),
        out_shape=jax.ShapeDtypeStruct((n, c), jnp.float32),
        compiler_params=pltpu.CompilerParams(dimension_semantics=("parallel",)),
    )(g, q, eq, pre, bd1, wd2, bd2, wg1, bg1, wg2, bg2, wo, bo)


def _tu_body(xf_ref, xct_ref, pc_ref, ff_ref, w1, b1, g1, t1,
             w2, b2, g2, t2, o_ref):
    o_ref[...] = _tu_core(xf_ref[...], xct_ref[...], pc_ref[...], ff_ref[...],
                          w1[...], b1[...], g1[...], t1[...],
                          w2[...], b2[...], g2[...], t2[...])


def _tu(p, xyzf8, xyzc8t, points_coarse, feats_fine):
    nf, c = feats_fine.shape
    s = points_coarse.shape[0]
    mf = min(256, nf)
    c2 = points_coarse.shape[1]
    r2 = lambda a: a.reshape(1, -1)
    return pl.pallas_call(
        _tu_body,
        grid=(nf // mf,),
        in_specs=[
            pl.BlockSpec((mf, 8), lambda i: (i, 0)),
            pl.BlockSpec((8, s), lambda i: (0, 0)),
            pl.BlockSpec((s, c2), lambda i: (0, 0)),
            pl.BlockSpec((mf, c), lambda i: (i, 0)),
        ] + [pl.BlockSpec(bs, lambda i: (0, 0)) for bs in
             [(c2, c), (1, c), (1, c), (1, c),
              (c, c), (1, c), (1, c), (1, c)]],
        out_specs=pl.BlockSpec((mf, c), lambda i: (i, 0)),
        out_shape=jax.ShapeDtypeStruct((nf, c), jnp.float32),
        compiler_params=pltpu.CompilerParams(dimension_semantics=("parallel",)),
    )(xyzf8, xyzc8t, points_coarse, feats_fine,
      p["fc1"]["w"], r2(p["fc1"]["b"]), r2(p["bn1"]["gamma"]), r2(p["bn1"]["beta"]),
      p["fc2"]["w"], r2(p["fc2"]["b"]), r2(p["bn2"]["gamma"]), r2(p["bn2"]["beta"]))


def _mlp_body(x_ref, w1, b1, w2, b2, w3, b3, o_ref):
    h = jnp.maximum(jnp.dot(x_ref[...], w1[...],
                            preferred_element_type=jnp.float32) + b1[...], 0.0)
    h = jnp.maximum(jnp.dot(h, w2[...],
                            preferred_element_type=jnp.float32) + b2[...], 0.0)
    o_ref[...] = jnp.dot(h, w3[...],
                         preferred_element_type=jnp.float32) + b3[...]


def _mlp(x, f):
    n, c = x.shape
    r2 = lambda a: a.reshape(1, -1)
    return pl.pallas_call(
        _mlp_body,
        out_shape=jax.ShapeDtypeStruct((n, c), jnp.float32),
    )(x, f["l1"]["w"], r2(f["l1"]["b"]), f["l2"]["w"], r2(f["l2"]["b"]),
      f["l3"]["w"], r2(f["l3"]["b"]))


def _f16_body(pts_ref, xyz_ref, l1w, l1b, l2w, l2b, l3w, l3b,
              wf, bf, wq, wk, wv, wd1, bd1, wd2, bd2,
              wg1, bg1, wg2, bg2, wo, bo, o_ref):
    dot = lambda a, b: jnp.dot(a, b, preferred_element_type=jnp.float32)
    n = 16
    hh = jnp.maximum(dot(pts_ref[...], l1w[...]) + l1b[...], 0.0)
    hh = jnp.maximum(dot(hh, l2w[...]) + l2b[...], 0.0)
    h = dot(hh, l3w[...]) + l3b[...]
    x = dot(h, wf[...]) + bf[...]
    q = dot(x, wq[...])
    kx = dot(x, wk[...])
    vx = dot(x, wv[...])
    e = dot(xyz_ref[...], wd1[...])
    qx = jnp.broadcast_to(q[:, None, :], (n, K, DM)).reshape(n * K, DM)
    eqx = jnp.broadcast_to(e[:, None, :], (n, K, DM)).reshape(n * K, DM)
    kk = jnp.broadcast_to(kx[None, :, :], (n, K, DM)).reshape(n * K, DM)
    vv = jnp.broadcast_to(vx[None, :, :], (n, K, DM)).reshape(n * K, DM)
    ee = jnp.broadcast_to(e[None, :, :], (n, K, DM)).reshape(n * K, DM)
    r = _attn_core(qx, eqx, kk, vv, ee, bd1[...], wd2[...], bd2[...],
                   wg1[...], bg1[...], wg2[...], bg2[...], n)
    o_ref[...] = dot(r, wo[...]) + bo[...] + h


def _f16(points, xyz32, f, p):
    n, c = points.shape
    r2 = lambda a: a.reshape(1, -1)
    wd1 = jnp.pad(p["delta1"]["w"], ((0, XP - 3), (0, 0)))
    return pl.pallas_call(
        _f16_body,
        out_shape=jax.ShapeDtypeStruct((n, c), jnp.float32),
    )(points, xyz32,
      f["l1"]["w"], r2(f["l1"]["b"]), f["l2"]["w"], r2(f["l2"]["b"]),
      f["l3"]["w"], r2(f["l3"]["b"]),
      p["fc1"]["w"], r2(p["fc1"]["b"]), p["wq"]["w"], p["wk"]["w"],
      p["wv"]["w"], wd1, r2(p["delta1"]["b"]), p["delta2"]["w"],
      r2(p["delta2"]["b"]), p["gamma1"]["w"], r2(p["gamma1"]["b"]),
      p["gamma2"]["w"], r2(p["gamma2"]["b"]), p["fc2"]["w"], r2(p["fc2"]["b"]))


def _fsm_body(xf_ref, xct_ref, pc_ref, ff_ref,
              w1, b1, g1, t1, w2, b2, g2, t2,
              xft_ref, xyz_ref, wf, bf, wq, wk, wv, wd1, bd1, wd2, bd2,
              wg1, bg1, wg2, bg2, wo, bo, o_ref, *, n):
    dot = lambda a, b: jnp.dot(a, b, preferred_element_type=jnp.float32)
    pts = _tu_core(xf_ref[...], xct_ref[...], pc_ref[...], ff_ref[...],
                   w1[...], b1[...], g1[...], t1[...],
                   w2[...], b2[...], g2[...], t2[...])
    d = _sqdist(xf_ref[...], xft_ref[...])
    idxc = jnp.concatenate(_topk_cols(d, K), axis=1)
    x = dot(pts, wf[...]) + bf[...]
    q = dot(x, wq[...])
    kx = dot(x, wk[...])
    vx = dot(x, wv[...])
    e = dot(xyz_ref[...], wd1[...])
    iota3 = jax.lax.broadcasted_iota(jnp.int32, (n, K, n), 2).astype(
        jnp.float32)
    oh = jnp.where(idxc[:, :, None] == iota3, 1.0, 0.0).reshape(n * K, n)
    kk = dot(oh, kx)
    vv = dot(oh, vx)
    ee = dot(oh, e)
    qx = jnp.broadcast_to(q[:, None, :], (n, K, DM)).reshape(n * K, DM)
    eqx = jnp.broadcast_to(e[:, None, :], (n, K, DM)).reshape(n * K, DM)
    r = _attn_core(qx, eqx, kk, vv, ee, bd1[...], wd2[...], bd2[...],
                   wg1[...], bg1[...], wg2[...], bg2[...], n)
    o_ref[...] = dot(r, wo[...]) + bo[...] + pts


def _fsm(xyzf8, xyzc8t, pc, ff, tu_p, xyzf8t, xyz32, tb_p):
    n, c = ff.shape
    r2 = lambda a: a.reshape(1, -1)
    wd1 = jnp.pad(tb_p["delta1"]["w"], ((0, XP - 3), (0, 0)))
    return pl.pallas_call(
        functools.partial(_fsm_body, n=n),
        out_shape=jax.ShapeDtypeStruct((n, c), jnp.float32),
    )(xyzf8, xyzc8t, pc, ff,
      tu_p["fc1"]["w"], r2(tu_p["fc1"]["b"]), r2(tu_p["bn1"]["gamma"]),
      r2(tu_p["bn1"]["beta"]), tu_p["fc2"]["w"], r2(tu_p["fc2"]["b"]),
      r2(tu_p["bn2"]["gamma"]), r2(tu_p["bn2"]["beta"]),
      xyzf8t, xyz32,
      tb_p["fc1"]["w"], r2(tb_p["fc1"]["b"]), tb_p["wq"]["w"],
      tb_p["wk"]["w"], tb_p["wv"]["w"], wd1, r2(tb_p["delta1"]["b"]),
      tb_p["delta2"]["w"], r2(tb_p["delta2"]["b"]),
      tb_p["gamma1"]["w"], r2(tb_p["gamma1"]["b"]),
      tb_p["gamma2"]["w"], r2(tb_p["gamma2"]["b"]),
      tb_p["fc2"]["w"], r2(tb_p["fc2"]["b"]))


def _tb(p, idx, xyz32, feats):
    r2 = lambda a: a.reshape(1, -1)
    wd1 = jnp.pad(p["delta1"]["w"], ((0, XP - 3), (0, 0)))
    q, e, table = _pre(feats, xyz32, p["fc1"]["w"], r2(p["fc1"]["b"]),
                       p["wq"]["w"], p["wk"]["w"], p["wv"]["w"], wd1)
    g = _sc_gather(table, idx.reshape(-1))
    return _attn(g, q, e, feats,
                 r2(p["delta1"]["b"]), p["delta2"]["w"], r2(p["delta2"]["b"]),
                 p["gamma1"]["w"], r2(p["gamma1"]["b"]),
                 p["gamma2"]["w"], r2(p["gamma2"]["b"]),
                 p["fc2"]["w"], r2(p["fc2"]["b"]))


def kernel(points, xyz0, xyz1, xyz2, xyz3, xyz4, feats0, feats1, feats2,
           feats3, params):
    xyzs = [x[0] for x in (xyz0, xyz1, xyz2, xyz3, xyz4)]
    featss = [f[0] for f in (feats0, feats1, feats2, feats3)]
    pad8 = [jnp.pad(x, ((0, 0), (0, 8 - 3))) for x in xyzs]
    pad8t = [x.T for x in pad8]
    pad32 = [jnp.pad(x, ((0, 0), (0, XP - 3))) for x in xyzs]

    idx1024 = _knn(pad8[1], pad8t[1], K)
    idx4096 = _knn(pad8[0], pad8t[0], K)

    pts = _f16(points[0], pad32[4], params["fc"], params["transformer"])
    pts = _fsm(pad8[3], pad8t[4], pts, featss[3], params["tu"][0],
               pad8t[3], pad32[3], params["tr"][0])
    pts = _fsm(pad8[2], pad8t[3], pts, featss[2], params["tu"][1],
               pad8t[2], pad32[2], params["tr"][1])

    pts = _tu(params["tu"][2], pad8[1], pad8t[2], pts, featss[1])
    pts = _tb(params["tr"][2], idx1024, pad32[1], pts)
    pts = _tu(params["tu"][3], pad8[0], pad8t[1], pts, featss[0])
    pts = _tb(params["tr"][3], idx4096, pad32[0], pts)
    return pts[None]
